# baseline (speedup 1.0000x reference)
import jax
import jax.numpy as jnp
from jax import lax
from jax.experimental import pallas as pl
from jax.experimental.pallas import tpu as pltpu
from jax.experimental.pallas import tpu_sc as plsc

NUM_ITEMS = 1000000
BATCH = 16384
EPB = BATCH // 16
NVEC = EPB // 16


def _sc_body(rating_h, item_h, titem_h, g_h,
             sh_tab, idx_s, idx_all, rat_v, val_v, z_v, g_v,
             sem1, sem2, sem3):
    cid = lax.axis_index("c")
    sid = lax.axis_index("s")

    e0 = sid * EPB
    c1 = pltpu.async_copy(item_h.at[pl.ds(e0, EPB)], idx_s, sem1)
    c2 = pltpu.async_copy(item_h.at[pl.ds(e0, EPB)],
                          idx_all.at[pl.ds(0, EPB)], sem2)
    c3 = pltpu.async_copy(titem_h.at[pl.ds(e0, EPB)],
                          idx_all.at[pl.ds(EPB, EPB)], sem2)
    c4 = pltpu.async_copy(rating_h.at[pl.ds(e0, EPB)], rat_v, sem3)

    zero16 = jnp.zeros((16,), jnp.float32)

    def zfill(i, _):
        z_v[pl.ds(i * 16, 16)] = zero16
        return 0

    lax.fori_loop(0, 2 * NVEC, zfill, 0)
    c4.wait()

    @pl.when(cid == 1)
    def _():
        def cfill(i, _):
            s = pl.ds(i * 16, 16)
            val_v[s] = jnp.where(rat_v[s] > 0.0, 1.0, 0.0)
            return 0
        lax.fori_loop(0, NVEC, cfill, 0)

    @pl.when(cid == 0)
    def _():
        def rfill(i, _):
            s = pl.ds(i * 16, 16)
            val_v[s] = rat_v[s]
            return 0
        lax.fori_loop(0, NVEC, rfill, 0)

    c1.wait()
    c2.wait()
    c3.wait()

    pltpu.sync_copy(z_v, sh_tab.at[idx_all])

    plsc.subcore_barrier()

    pltpu.sync_copy(val_v, sh_tab.at[idx_s], add=True)

    plsc.subcore_barrier()

    pltpu.sync_copy(sh_tab.at[idx_all], g_v)
    o1 = pltpu.async_copy(g_v.at[pl.ds(0, EPB)],
                          g_h.at[cid, 0, pl.ds(e0, EPB)], sem1)
    o2 = pltpu.async_copy(g_v.at[pl.ds(EPB, EPB)],
                          g_h.at[cid, 1, pl.ds(e0, EPB)], sem2)
    o1.wait()
    o2.wait()


def _tc_epilogue(g_ref, rt_ref, tr_ref, pred_ref, loss_ref):
    gb = g_ref[0:128, :]
    gc = g_ref[256:384, :]
    sel = rt_ref[...] > 0.0
    ceff = jnp.where(sel, gc, 1.0)
    num = jnp.sum(jnp.where(sel, gb / (ceff * ceff), 0.0))
    nnz = jnp.sum(jnp.where(sel, 1.0 / ceff, 0.0))
    fb = num / jnp.maximum(nnz, 1.0)

    bt = g_ref[128:256, :]
    ct = g_ref[384:512, :]
    pred = jnp.where(ct == 0.0, fb, bt / (ct + 1e-10))
    pred_ref[...] = pred
    err = pred - tr_ref[...]
    loss_ref[...] = (jnp.sum(err * err) * (1.0 / BATCH)).reshape(1, 1)


def kernel(rating, item, target_rating, target_item, base, count):
    item = item.astype(jnp.int32)
    target_item = target_item.astype(jnp.int32)

    sc_call = pl.kernel(
        _sc_body,
        out_type=[
            jax.ShapeDtypeStruct((2, 2, BATCH), jnp.float32),
        ],
        scratch_types=[
            pltpu.VMEM_SHARED((NUM_ITEMS,), jnp.float32),
            pltpu.VMEM((EPB,), jnp.int32),
            pltpu.VMEM((2 * EPB,), jnp.int32),
            pltpu.VMEM((EPB,), jnp.float32),
            pltpu.VMEM((EPB,), jnp.float32),
            pltpu.VMEM((2 * EPB,), jnp.float32),
            pltpu.VMEM((2 * EPB,), jnp.float32),
            pltpu.SemaphoreType.DMA,
            pltpu.SemaphoreType.DMA,
            pltpu.SemaphoreType.DMA,
        ],
        mesh=plsc.VectorSubcoreMesh(core_axis_name="c", subcore_axis_name="s"),
    )
    (g,) = sc_call(rating, item, target_item)

    pred2, loss2 = pl.pallas_call(
        _tc_epilogue,
        out_shape=[
            jax.ShapeDtypeStruct((128, 128), jnp.float32),
            jax.ShapeDtypeStruct((1, 1), jnp.float32),
        ],
    )(g.reshape(512, 128), rating.reshape(128, 128),
      target_rating.reshape(128, 128))

    return pred2.reshape(BATCH), loss2[0, 0]
